# Initial kernel scaffold; baseline (speedup 1.0000x reference)
import jax
import jax.numpy as jnp
from jax.experimental import pallas as pl


def kernel(inputs, Wf, bf, Wi, bi, Wg, bg, Wo, bo):
    raise NotImplementedError("write your pallas kernel here")


# speedup vs baseline: 2.0188x; 2.0188x over previous
import jax
import jax.numpy as jnp
from jax.experimental import pallas as pl
from jax.experimental.pallas import tpu as pltpu

T, B, D_IN, D_H = 512, 64, 512, 512
G4 = 4 * D_H
BM = 1024
NC = 2
BC = B // NC


def _proj_kernel(x_ref, wxt_ref, b_ref, o_ref):
    o_ref[...] = (
        jnp.dot(x_ref[...], wxt_ref[...], preferred_element_type=jnp.float32)
        + b_ref[...]
    )


def _rec_kernel(pre_ref, wht_ref, out_ref, cx_ref, h_ref, c_ref):
    t = pl.program_id(1)

    @pl.when(t == 0)
    def _():
        h_ref[...] = jnp.zeros_like(h_ref)
        c_ref[...] = jnp.zeros_like(c_ref)

    gates = pre_ref[0] + jnp.dot(
        h_ref[...], wht_ref[...], preferred_element_type=jnp.float32
    )
    f = jax.nn.sigmoid(gates[:, 0 * D_H : 1 * D_H])
    i = jax.nn.sigmoid(gates[:, 1 * D_H : 2 * D_H])
    g = jnp.tanh(gates[:, 2 * D_H : 3 * D_H])
    o = jax.nn.sigmoid(gates[:, 3 * D_H : 4 * D_H])
    c_new = f * c_ref[...] + i * g
    h_new = o * jnp.tanh(c_new)
    c_ref[...] = c_new
    h_ref[...] = h_new
    out_ref[0] = h_new

    @pl.when(t == T - 1)
    def _():
        cx_ref[...] = c_new


def kernel(inputs, Wf, bf, Wi, bi, Wg, bg, Wo, bo):
    W = jnp.concatenate([Wf, Wi, Wg, Wo], axis=0)
    WxT = W[:, :D_IN].T
    WhT = W[:, D_IN:].T
    b = jnp.concatenate([bf, bi, bg, bo]).reshape(1, G4)

    X = inputs.reshape(T * B, D_IN)
    pre = pl.pallas_call(
        _proj_kernel,
        out_shape=jax.ShapeDtypeStruct((T * B, G4), jnp.float32),
        grid=(T * B // BM,),
        in_specs=[
            pl.BlockSpec((BM, D_IN), lambda m: (m, 0)),
            pl.BlockSpec((D_IN, G4), lambda m: (0, 0)),
            pl.BlockSpec((1, G4), lambda m: (0, 0)),
        ],
        out_specs=pl.BlockSpec((BM, G4), lambda m: (m, 0)),
        compiler_params=pltpu.CompilerParams(
            dimension_semantics=("parallel",),
        ),
        name="lstm_proj",
    )(X, WxT, b)
    pre = pre.reshape(T, B, G4)

    outputs, cx = pl.pallas_call(
        _rec_kernel,
        out_shape=(
            jax.ShapeDtypeStruct((T, B, D_H), jnp.float32),
            jax.ShapeDtypeStruct((B, D_H), jnp.float32),
        ),
        grid=(NC, T),
        in_specs=[
            pl.BlockSpec((1, BC, G4), lambda c, t: (t, c, 0)),
            pl.BlockSpec((D_H, G4), lambda c, t: (0, 0)),
        ],
        out_specs=(
            pl.BlockSpec((1, BC, D_H), lambda c, t: (t, c, 0)),
            pl.BlockSpec((BC, D_H), lambda c, t: (c, 0)),
        ),
        scratch_shapes=[
            pltpu.VMEM((BC, D_H), jnp.float32),
            pltpu.VMEM((BC, D_H), jnp.float32),
        ],
        compiler_params=pltpu.CompilerParams(
            dimension_semantics=("parallel", "arbitrary"),
        ),
        name="lstm_rec",
    )(pre, WhT)

    hx = outputs[-1]
    return outputs, (hx, cx)
